# baseline (speedup 1.0000x reference)
import functools

import jax
import jax.numpy as jnp
from jax import lax
from jax.experimental import pallas as pl
from jax.experimental.pallas import tpu as pltpu
from jax.experimental.pallas import tpu_sc as plsc

N = 10000
E = 320000
D = 128
G = 64
NC = 2
NS = 16
CHUNK = 128
EPW = 10240
NPADE = EPW - E // (NC * NS)
NHALF = 2
NCH = EPW // CHUNK // NHALF
NPAIR = NCH // 2
NPAD = 10240
RPW = NPAD // NS
ZROWS = 128


@functools.cache
def _make_sc_agg():
    mesh = plsc.VectorSubcoreMesh(core_axis_name="c", subcore_axis_name="s",
                                  num_cores=NC, num_subcores=NS)

    @functools.partial(
        pl.kernel,
        out_type=jax.ShapeDtypeStruct((NC, NPAD, D), jnp.float32),
        mesh=mesh,
        scratch_types=[
            pltpu.VMEM_SHARED((NPAD, D), jnp.float32),
            pltpu.VMEM((NCH, CHUNK), jnp.int32),
            pltpu.VMEM((NCH, CHUNK), jnp.int32),
            pltpu.VMEM((CHUNK, D), jnp.float32),
            pltpu.VMEM((CHUNK, D), jnp.float32),
            pltpu.SemaphoreType.DMA,
            pltpu.SemaphoreType.DMA,
        ],
    )
    def sc_agg(h_hbm, edges_hbm, out_hbm,
               acc_sh, src_v, dst_v, rows0_v, rows1_v, sem0, sem1):
        c = lax.axis_index("c")
        s = lax.axis_index("s")

        def _zrow(i, _):
            for k in range(D // 16):
                rows0_v[i, pl.ds(k * 16, 16)] = jnp.zeros((16,), jnp.float32)
            return 0
        lax.fori_loop(0, ZROWS, _zrow, 0)
        for t in range(RPW // ZROWS):
            pltpu.sync_copy(rows0_v.at[pl.ds(0, ZROWS)],
                            acc_sh.at[pl.ds(s * RPW + t * ZROWS, ZROWS)])
        plsc.subcore_barrier()

        for half in range(NHALF):
            pltpu.sync_copy(edges_hbm.at[0, c, s, half], src_v)
            pltpu.sync_copy(edges_hbm.at[1, c, s, half], dst_v)
            pltpu.async_copy(h_hbm.at[src_v.at[0]], rows0_v, sem0)
            pltpu.async_copy(h_hbm.at[src_v.at[1]], rows1_v, sem1)

            def _pair(jj, _):
                j0 = 2 * jj
                pltpu.make_async_copy(h_hbm.at[src_v.at[j0]], rows0_v,
                                      sem0).wait()
                pltpu.sync_copy(rows0_v, acc_sh.at[dst_v.at[j0]], add=True)

                @pl.when(jj + 1 < NPAIR)
                def _():
                    pltpu.async_copy(h_hbm.at[src_v.at[j0 + 2]], rows0_v,
                                     sem0)

                pltpu.make_async_copy(h_hbm.at[src_v.at[j0 + 1]], rows1_v,
                                      sem1).wait()
                pltpu.sync_copy(rows1_v, acc_sh.at[dst_v.at[j0 + 1]],
                                add=True)

                @pl.when(jj + 1 < NPAIR)
                def _():
                    pltpu.async_copy(h_hbm.at[src_v.at[j0 + 3]], rows1_v,
                                     sem1)

                return 0
            lax.fori_loop(0, NPAIR, _pair, 0)
        plsc.subcore_barrier()

        pltpu.sync_copy(acc_sh.at[pl.ds(s * RPW, RPW)],
                        out_hbm.at[c, pl.ds(s * RPW, RPW)])

    return sc_agg


ROWS_BLK = 2048


def _tc_layer_body(h_ref, a_ref, b_ref, w1_ref, b1_ref,
                   w2_ref, b2_ref, o_ref, ro_ref):
    hv = h_ref[...] + a_ref[0] + a_ref[1]
    t = jnp.maximum(
        jnp.dot(hv, w1_ref[...], preferred_element_type=jnp.float32)
        + b1_ref[...], 0.0)
    o = jnp.dot(t, w2_ref[...], preferred_element_type=jnp.float32) + b2_ref[...]
    hn = jnp.maximum(o, 0.0)
    o_ref[...] = hn
    bb = b_ref[pl.ds(pl.program_id(0) * ROWS_BLK, ROWS_BLK)]
    sel = (lax.broadcasted_iota(jnp.int32, (G, ROWS_BLK), 0)
           == bb[None, :]).astype(jnp.float32)
    contrib = jnp.dot(sel, hn, preferred_element_type=jnp.float32)

    @pl.when(pl.program_id(0) == 0)
    def _():
        ro_ref[...] = contrib

    @pl.when(pl.program_id(0) != 0)
    def _():
        ro_ref[...] += contrib


def _tc_layer(h, agg, batch, w1, b1, w2, b2):
    nblk = NPAD // ROWS_BLK
    return pl.pallas_call(
        _tc_layer_body,
        grid=(nblk,),
        in_specs=[
            pl.BlockSpec((ROWS_BLK, D), lambda i: (i, 0)),
            pl.BlockSpec((NC, ROWS_BLK, D), lambda i: (0, i, 0)),
            pl.BlockSpec((NPAD,), lambda i: (0,)),
            pl.BlockSpec((D, D), lambda i: (0, 0)),
            pl.BlockSpec((1, D), lambda i: (0, 0)),
            pl.BlockSpec((D, D), lambda i: (0, 0)),
            pl.BlockSpec((1, D), lambda i: (0, 0)),
        ],
        out_specs=[
            pl.BlockSpec((ROWS_BLK, D), lambda i: (i, 0)),
            pl.BlockSpec((G, D), lambda i: (0, 0)),
        ],
        out_shape=[
            jax.ShapeDtypeStruct((NPAD, D), jnp.float32),
            jax.ShapeDtypeStruct((G, D), jnp.float32),
        ],
    )(h, agg, batch, w1, b1, w2, b2)


def kernel(x, edge_index, batch, device, gin_params, mlp_params, pool_params):
    del device, mlp_params, pool_params
    e3 = edge_index.astype(jnp.int32).reshape(2, NC * NS, E // (NC * NS))
    pad_dst = jnp.broadcast_to(N + jnp.arange(NPADE, dtype=jnp.int32),
                               (NC * NS, NPADE))
    pads = jnp.stack([jnp.zeros((NC * NS, NPADE), jnp.int32), pad_dst])
    edges = jnp.concatenate([e3, pads], axis=2).reshape(
        2, NC, NS, NHALF, NCH, CHUNK)
    batch = jnp.full((NPAD,), G, jnp.int32).at[:N].set(batch.astype(jnp.int32))

    h = jnp.zeros((NPAD, D), x.dtype).at[:N].set(x)
    readouts = []
    for p in gin_params:
        agg = _make_sc_agg()(h, edges)
        h, ro = _tc_layer(h, agg, batch,
                          p["W1"], p["b1"].reshape(1, D),
                          p["W2"], p["b2"].reshape(1, D))
        readouts.append(ro)
    m1p = jnp.concatenate(readouts, axis=1)
    return (m1p, m1p)


# speedup vs baseline: 1.8666x; 1.8666x over previous
import functools

import jax
import jax.numpy as jnp
from jax import lax
from jax.experimental import pallas as pl
from jax.experimental.pallas import tpu as pltpu
from jax.experimental.pallas import tpu_sc as plsc

N = 10000
E = 320000
D = 128
G = 64
NC = 2
NS = 16
CHUNK = 120
EPW = 10080
NPADE = EPW - E // (NC * NS)
NHALF = 2
NCH = EPW // CHUNK // NHALF
NPAIR = NCH // 2
NPAD = 10240
RPW = NPAD // NS
ZROWS = 128


@functools.cache
def _make_sc_agg():
    mesh = plsc.VectorSubcoreMesh(core_axis_name="c", subcore_axis_name="s",
                                  num_cores=NC, num_subcores=NS)

    @functools.partial(
        pl.kernel,
        out_type=jax.ShapeDtypeStruct((NC, NPAD, D), jnp.float32),
        mesh=mesh,
        scratch_types=[
            pltpu.VMEM_SHARED((NPAD, D), jnp.float32),
            pltpu.VMEM((NCH, CHUNK), jnp.int32),
            pltpu.VMEM((NCH, CHUNK), jnp.int32),
            pltpu.VMEM((CHUNK, D), jnp.float32),
            pltpu.VMEM((CHUNK, D), jnp.float32),
            pltpu.SemaphoreType.DMA,
            pltpu.SemaphoreType.DMA,
        ],
    )
    def sc_agg(h_hbm, edges_hbm, out_hbm,
               acc_sh, src_v, dst_v, rows0_v, rows1_v, sem0, sem1):
        c = lax.axis_index("c")
        s = lax.axis_index("s")

        def _zrow(i, _):
            for k in range(D // 16):
                rows0_v[i, pl.ds(k * 16, 16)] = jnp.zeros((16,), jnp.float32)
            return 0
        lax.fori_loop(0, ZROWS, _zrow, 0)
        for t in range(RPW // ZROWS):
            pltpu.sync_copy(rows0_v.at[pl.ds(0, ZROWS)],
                            acc_sh.at[pl.ds(s * RPW + t * ZROWS, ZROWS)])
        plsc.subcore_barrier()

        for half in range(NHALF):
            pltpu.sync_copy(edges_hbm.at[0, c, s, half], src_v)
            pltpu.sync_copy(edges_hbm.at[1, c, s, half], dst_v)
            pltpu.async_copy(h_hbm.at[src_v.at[0]], rows0_v, sem0)
            pltpu.async_copy(h_hbm.at[src_v.at[1]], rows1_v, sem1)

            def _pair(jj, _):
                j0 = 2 * jj
                pltpu.make_async_copy(h_hbm.at[src_v.at[j0]], rows0_v,
                                      sem0).wait()
                pltpu.sync_copy(rows0_v, acc_sh.at[dst_v.at[j0]], add=True)

                @pl.when(jj + 1 < NPAIR)
                def _():
                    pltpu.async_copy(h_hbm.at[src_v.at[j0 + 2]], rows0_v,
                                     sem0)

                pltpu.make_async_copy(h_hbm.at[src_v.at[j0 + 1]], rows1_v,
                                      sem1).wait()
                pltpu.sync_copy(rows1_v, acc_sh.at[dst_v.at[j0 + 1]],
                                add=True)

                @pl.when(jj + 1 < NPAIR)
                def _():
                    pltpu.async_copy(h_hbm.at[src_v.at[j0 + 3]], rows1_v,
                                     sem1)

                return 0
            lax.fori_loop(0, NPAIR, _pair, 0)
        plsc.subcore_barrier()

        pltpu.sync_copy(acc_sh.at[pl.ds(s * RPW, RPW)],
                        out_hbm.at[c, pl.ds(s * RPW, RPW)])

    return sc_agg


ROWS_BLK = 2048


def _tc_layer_body(h_ref, a_ref, b_ref, w1_ref, b1_ref,
                   w2_ref, b2_ref, o_ref, ro_ref):
    hv = h_ref[...] + a_ref[0] + a_ref[1]
    t = jnp.maximum(
        jnp.dot(hv, w1_ref[...], preferred_element_type=jnp.float32)
        + b1_ref[...], 0.0)
    o = jnp.dot(t, w2_ref[...], preferred_element_type=jnp.float32) + b2_ref[...]
    hn = jnp.maximum(o, 0.0)
    o_ref[...] = hn
    bb = b_ref[pl.ds(pl.program_id(0) * ROWS_BLK, ROWS_BLK)]
    sel = (lax.broadcasted_iota(jnp.int32, (G, ROWS_BLK), 0)
           == bb[None, :]).astype(jnp.float32)
    contrib = jnp.dot(sel, hn, preferred_element_type=jnp.float32)

    @pl.when(pl.program_id(0) == 0)
    def _():
        ro_ref[...] = contrib

    @pl.when(pl.program_id(0) != 0)
    def _():
        ro_ref[...] += contrib


def _tc_layer(h, agg, batch, w1, b1, w2, b2):
    nblk = NPAD // ROWS_BLK
    return pl.pallas_call(
        _tc_layer_body,
        grid=(nblk,),
        in_specs=[
            pl.BlockSpec((ROWS_BLK, D), lambda i: (i, 0)),
            pl.BlockSpec((NC, ROWS_BLK, D), lambda i: (0, i, 0)),
            pl.BlockSpec((NPAD,), lambda i: (0,)),
            pl.BlockSpec((D, D), lambda i: (0, 0)),
            pl.BlockSpec((1, D), lambda i: (0, 0)),
            pl.BlockSpec((D, D), lambda i: (0, 0)),
            pl.BlockSpec((1, D), lambda i: (0, 0)),
        ],
        out_specs=[
            pl.BlockSpec((ROWS_BLK, D), lambda i: (i, 0)),
            pl.BlockSpec((G, D), lambda i: (0, 0)),
        ],
        out_shape=[
            jax.ShapeDtypeStruct((NPAD, D), jnp.float32),
            jax.ShapeDtypeStruct((G, D), jnp.float32),
        ],
    )(h, agg, batch, w1, b1, w2, b2)


def kernel(x, edge_index, batch, device, gin_params, mlp_params, pool_params):
    del device, mlp_params, pool_params
    e3 = edge_index.astype(jnp.int32).reshape(2, NC * NS, E // (NC * NS))
    pad_dst = jnp.broadcast_to(N + jnp.arange(NPADE, dtype=jnp.int32),
                               (NC * NS, NPADE))
    pads = jnp.stack([jnp.zeros((NC * NS, NPADE), jnp.int32), pad_dst])
    edges = jnp.concatenate([e3, pads], axis=2).reshape(
        2, NC, NS, NHALF, NCH, CHUNK)
    batch = jnp.full((NPAD,), G, jnp.int32).at[:N].set(batch.astype(jnp.int32))

    h = jnp.zeros((NPAD, D), x.dtype).at[:N].set(x)
    readouts = []
    for p in gin_params:
        agg = _make_sc_agg()(h, edges)
        h, ro = _tc_layer(h, agg, batch,
                          p["W1"], p["b1"].reshape(1, D),
                          p["W2"], p["b2"].reshape(1, D))
        readouts.append(ro)
    m1p = jnp.concatenate(readouts, axis=1)
    return (m1p, m1p)


# speedup vs baseline: 3.2072x; 1.7182x over previous
import functools

import jax
import jax.numpy as jnp
from jax import lax
from jax.experimental import pallas as pl
from jax.experimental.pallas import tpu as pltpu
from jax.experimental.pallas import tpu_sc as plsc

N = 10000
E = 320000
D = 128
G = 64
NC = 2
NS = 16
CHUNK = 128
EPW = 10240
NPADE = EPW - E // (NC * NS)
NHALF = 2
NCH = EPW // CHUNK // NHALF
NPAIR = NCH // 2
NPAD = 10240
RPW = NPAD // NS
ZROWS = 128


@functools.cache
def _make_sc_agg():
    mesh = plsc.VectorSubcoreMesh(core_axis_name="c", subcore_axis_name="s",
                                  num_cores=NC, num_subcores=NS)

    @functools.partial(
        pl.kernel,
        out_type=jax.ShapeDtypeStruct((NC, NPAD, D), jnp.float32),
        mesh=mesh,
        scratch_types=[
            pltpu.VMEM_SHARED((NPAD, D), jnp.float32),
            pltpu.VMEM((NCH, CHUNK), jnp.int32),
            pltpu.VMEM((NCH, CHUNK), jnp.int32),
            pltpu.VMEM((CHUNK, D), jnp.float32),
            pltpu.VMEM((CHUNK, D), jnp.float32),
            pltpu.SemaphoreType.DMA,
            pltpu.SemaphoreType.DMA,
        ],
    )
    def sc_agg(h_hbm, edges_hbm, out_hbm,
               acc_sh, src_v, dst_v, rows0_v, rows1_v, sem0, sem1):
        c = lax.axis_index("c")
        s = lax.axis_index("s")

        def _zrow(i, _):
            for k in range(D // 16):
                rows0_v[i, pl.ds(k * 16, 16)] = jnp.zeros((16,), jnp.float32)
            return 0
        lax.fori_loop(0, ZROWS, _zrow, 0)
        for t in range(RPW // ZROWS):
            pltpu.sync_copy(rows0_v.at[pl.ds(0, ZROWS)],
                            acc_sh.at[pl.ds(s * RPW + t * ZROWS, ZROWS)])
        plsc.subcore_barrier()

        for half in range(NHALF):
            pltpu.sync_copy(edges_hbm.at[0, c, s, half], src_v)
            pltpu.sync_copy(edges_hbm.at[1, c, s, half], dst_v)
            pltpu.async_copy(h_hbm.at[src_v.at[0]], rows0_v, sem0)
            pltpu.async_copy(h_hbm.at[src_v.at[1]], rows1_v, sem1)

            def _pair(jj, _):
                j0 = 2 * jj
                pltpu.make_async_copy(h_hbm.at[src_v.at[j0]], rows0_v,
                                      sem0).wait()
                pltpu.sync_copy(rows0_v, acc_sh.at[dst_v.at[j0]], add=True)

                @pl.when(jj + 1 < NPAIR)
                def _():
                    pltpu.async_copy(h_hbm.at[src_v.at[j0 + 2]], rows0_v,
                                     sem0)

                pltpu.make_async_copy(h_hbm.at[src_v.at[j0 + 1]], rows1_v,
                                      sem1).wait()
                pltpu.sync_copy(rows1_v, acc_sh.at[dst_v.at[j0 + 1]],
                                add=True)

                @pl.when(jj + 1 < NPAIR)
                def _():
                    pltpu.async_copy(h_hbm.at[src_v.at[j0 + 3]], rows1_v,
                                     sem1)

                return 0
            lax.fori_loop(0, NPAIR, _pair, 0)
        plsc.subcore_barrier()

        pltpu.sync_copy(acc_sh.at[pl.ds(s * RPW, RPW)],
                        out_hbm.at[c, pl.ds(s * RPW, RPW)])

    return sc_agg


ROWS_BLK = 2048


def _tc_layer_body(h_ref, a_ref, b_ref, w1_ref, b1_ref,
                   w2_ref, b2_ref, o_ref, ro_ref):
    hv = h_ref[...] + a_ref[0] + a_ref[1]
    t = jnp.maximum(
        jnp.dot(hv, w1_ref[...], preferred_element_type=jnp.float32)
        + b1_ref[...], 0.0)
    o = jnp.dot(t, w2_ref[...], preferred_element_type=jnp.float32) + b2_ref[...]
    hn = jnp.maximum(o, 0.0)
    rid = (lax.broadcasted_iota(jnp.int32, (ROWS_BLK, 1), 0)
           + pl.program_id(0) * ROWS_BLK)
    hn = jnp.where(rid < N, hn, 0.0)
    o_ref[...] = hn
    bb = b_ref[pl.ds(pl.program_id(0) * ROWS_BLK, ROWS_BLK)]
    sel = (lax.broadcasted_iota(jnp.int32, (G, ROWS_BLK), 0)
           == bb[None, :]).astype(jnp.float32)
    contrib = jnp.dot(sel, hn, preferred_element_type=jnp.float32)

    @pl.when(pl.program_id(0) == 0)
    def _():
        ro_ref[...] = contrib

    @pl.when(pl.program_id(0) != 0)
    def _():
        ro_ref[...] += contrib


def _tc_layer(h, agg, batch, w1, b1, w2, b2):
    nblk = NPAD // ROWS_BLK
    return pl.pallas_call(
        _tc_layer_body,
        grid=(nblk,),
        in_specs=[
            pl.BlockSpec((ROWS_BLK, D), lambda i: (i, 0)),
            pl.BlockSpec((NC, ROWS_BLK, D), lambda i: (0, i, 0)),
            pl.BlockSpec((NPAD,), lambda i: (0,)),
            pl.BlockSpec((D, D), lambda i: (0, 0)),
            pl.BlockSpec((1, D), lambda i: (0, 0)),
            pl.BlockSpec((D, D), lambda i: (0, 0)),
            pl.BlockSpec((1, D), lambda i: (0, 0)),
        ],
        out_specs=[
            pl.BlockSpec((ROWS_BLK, D), lambda i: (i, 0)),
            pl.BlockSpec((G, D), lambda i: (0, 0)),
        ],
        out_shape=[
            jax.ShapeDtypeStruct((NPAD, D), jnp.float32),
            jax.ShapeDtypeStruct((G, D), jnp.float32),
        ],
    )(h, agg, batch, w1, b1, w2, b2)


def kernel(x, edge_index, batch, device, gin_params, mlp_params, pool_params):
    del device, mlp_params, pool_params
    nw = NC * NS
    e3 = edge_index.astype(jnp.int32).reshape(2, nw, E // nw)
    wids = jnp.arange(nw, dtype=jnp.int32)[:, None]
    ii = jnp.arange(NPADE, dtype=jnp.int32)[None, :]
    pad_src = jnp.broadcast_to(N + (ii % (NPAD - N)), (nw, NPADE))
    pad_dst = jnp.broadcast_to(wids + nw * ii, (nw, NPADE))
    pads = jnp.stack([pad_src, pad_dst])
    edges = jnp.concatenate([e3, pads], axis=2).reshape(
        2, NC, NS, NHALF, NCH, CHUNK)
    batch = jnp.full((NPAD,), G, jnp.int32).at[:N].set(batch.astype(jnp.int32))

    h = jnp.zeros((NPAD, D), x.dtype).at[:N].set(x)
    readouts = []
    for p in gin_params:
        agg = _make_sc_agg()(h, edges)
        h, ro = _tc_layer(h, agg, batch,
                          p["W1"], p["b1"].reshape(1, D),
                          p["W2"], p["b2"].reshape(1, D))
        readouts.append(ro)
    m1p = jnp.concatenate(readouts, axis=1)
    return (m1p, m1p)


# speedup vs baseline: 3.2561x; 1.0153x over previous
import functools

import jax
import jax.numpy as jnp
from jax import lax
from jax.experimental import pallas as pl
from jax.experimental.pallas import tpu as pltpu
from jax.experimental.pallas import tpu_sc as plsc

N = 10000
E = 320000
D = 128
G = 64
NC = 2
NS = 16
CHUNK = 128
EPW = 10240
NPADE = EPW - E // (NC * NS)
NHALF = 2
NCH = EPW // CHUNK // NHALF
NPAIR = NCH // 2
NPAD = 10240
RPW = NPAD // NS
ZROWS = 128


@functools.cache
def _make_sc_agg():
    mesh = plsc.VectorSubcoreMesh(core_axis_name="c", subcore_axis_name="s",
                                  num_cores=NC, num_subcores=NS)

    @functools.partial(
        pl.kernel,
        out_type=jax.ShapeDtypeStruct((NC, NPAD, D), jnp.float32),
        mesh=mesh,
        scratch_types=[
            pltpu.VMEM_SHARED((NPAD, D), jnp.float32),
            pltpu.VMEM((NCH, CHUNK), jnp.int32),
            pltpu.VMEM((NCH, CHUNK), jnp.int32),
            pltpu.VMEM((CHUNK, D), jnp.float32),
            pltpu.VMEM((CHUNK, D), jnp.float32),
            pltpu.SemaphoreType.DMA,
            pltpu.SemaphoreType.DMA,
        ],
    )
    def sc_agg(h_hbm, edges_hbm, out_hbm,
               acc_sh, src_v, dst_v, rows0_v, rows1_v, sem0, sem1):
        c = lax.axis_index("c")
        s = lax.axis_index("s")

        def _zrow(i, _):
            for k in range(D // 16):
                rows0_v[i, pl.ds(k * 16, 16)] = jnp.zeros((16,), jnp.float32)
            return 0
        lax.fori_loop(0, ZROWS, _zrow, 0)
        for t in range(RPW // ZROWS):
            pltpu.async_copy(rows0_v.at[pl.ds(0, ZROWS)],
                             acc_sh.at[pl.ds(s * RPW + t * ZROWS, ZROWS)],
                             sem1)
        pltpu.sync_copy(edges_hbm.at[0, c, s, 0], src_v)
        pltpu.sync_copy(edges_hbm.at[1, c, s, 0], dst_v)
        for t in range(RPW // ZROWS):
            pltpu.make_async_copy(
                rows0_v.at[pl.ds(0, ZROWS)],
                acc_sh.at[pl.ds(s * RPW + t * ZROWS, ZROWS)], sem1).wait()
        plsc.subcore_barrier()

        for half in range(NHALF):
            if half > 0:
                pltpu.sync_copy(edges_hbm.at[0, c, s, half], src_v)
                pltpu.sync_copy(edges_hbm.at[1, c, s, half], dst_v)
            pltpu.async_copy(h_hbm.at[src_v.at[0]], rows0_v, sem0)
            pltpu.async_copy(h_hbm.at[src_v.at[1]], rows1_v, sem1)

            def _pair(jj, _):
                j0 = 2 * jj
                pltpu.make_async_copy(h_hbm.at[src_v.at[j0]], rows0_v,
                                      sem0).wait()
                pltpu.sync_copy(rows0_v, acc_sh.at[dst_v.at[j0]], add=True)

                @pl.when(jj + 1 < NPAIR)
                def _():
                    pltpu.async_copy(h_hbm.at[src_v.at[j0 + 2]], rows0_v,
                                     sem0)

                pltpu.make_async_copy(h_hbm.at[src_v.at[j0 + 1]], rows1_v,
                                      sem1).wait()
                pltpu.sync_copy(rows1_v, acc_sh.at[dst_v.at[j0 + 1]],
                                add=True)

                @pl.when(jj + 1 < NPAIR)
                def _():
                    pltpu.async_copy(h_hbm.at[src_v.at[j0 + 3]], rows1_v,
                                     sem1)

                return 0
            lax.fori_loop(0, NPAIR, _pair, 0)
        plsc.subcore_barrier()

        pltpu.sync_copy(acc_sh.at[pl.ds(s * RPW, RPW)],
                        out_hbm.at[c, pl.ds(s * RPW, RPW)])

    return sc_agg


ROWS_BLK = 2048


def _tc_layer_body(h_ref, a_ref, b_ref, w1_ref, b1_ref,
                   w2_ref, b2_ref, o_ref, ro_ref):
    hv = h_ref[...] + a_ref[0] + a_ref[1]
    t = jnp.maximum(
        jnp.dot(hv, w1_ref[...], preferred_element_type=jnp.float32)
        + b1_ref[...], 0.0)
    o = jnp.dot(t, w2_ref[...], preferred_element_type=jnp.float32) + b2_ref[...]
    hn = jnp.maximum(o, 0.0)
    rid = (lax.broadcasted_iota(jnp.int32, (ROWS_BLK, 1), 0)
           + pl.program_id(0) * ROWS_BLK)
    hn = jnp.where(rid < N, hn, 0.0)
    o_ref[...] = hn
    bb = b_ref[pl.ds(pl.program_id(0) * ROWS_BLK, ROWS_BLK)]
    sel = (lax.broadcasted_iota(jnp.int32, (G, ROWS_BLK), 0)
           == bb[None, :]).astype(jnp.float32)
    contrib = jnp.dot(sel, hn, preferred_element_type=jnp.float32)

    @pl.when(pl.program_id(0) == 0)
    def _():
        ro_ref[...] = contrib

    @pl.when(pl.program_id(0) != 0)
    def _():
        ro_ref[...] += contrib


def _tc_layer(h, agg, batch, w1, b1, w2, b2):
    nblk = NPAD // ROWS_BLK
    return pl.pallas_call(
        _tc_layer_body,
        grid=(nblk,),
        in_specs=[
            pl.BlockSpec((ROWS_BLK, D), lambda i: (i, 0)),
            pl.BlockSpec((NC, ROWS_BLK, D), lambda i: (0, i, 0)),
            pl.BlockSpec((NPAD,), lambda i: (0,)),
            pl.BlockSpec((D, D), lambda i: (0, 0)),
            pl.BlockSpec((1, D), lambda i: (0, 0)),
            pl.BlockSpec((D, D), lambda i: (0, 0)),
            pl.BlockSpec((1, D), lambda i: (0, 0)),
        ],
        out_specs=[
            pl.BlockSpec((ROWS_BLK, D), lambda i: (i, 0)),
            pl.BlockSpec((G, D), lambda i: (0, 0)),
        ],
        out_shape=[
            jax.ShapeDtypeStruct((NPAD, D), jnp.float32),
            jax.ShapeDtypeStruct((G, D), jnp.float32),
        ],
    )(h, agg, batch, w1, b1, w2, b2)


def kernel(x, edge_index, batch, device, gin_params, mlp_params, pool_params):
    del device, mlp_params, pool_params
    nw = NC * NS
    e3 = edge_index.astype(jnp.int32).reshape(2, nw, E // nw)
    wids = jnp.arange(nw, dtype=jnp.int32)[:, None]
    ii = jnp.arange(NPADE, dtype=jnp.int32)[None, :]
    pad_src = jnp.broadcast_to(N + (ii % (NPAD - N)), (nw, NPADE))
    pad_dst = jnp.broadcast_to(wids + nw * ii, (nw, NPADE))
    pads = jnp.stack([pad_src, pad_dst])
    edges = jnp.concatenate([e3, pads], axis=2).reshape(
        2, NC, NS, NHALF, NCH, CHUNK)
    batch = jnp.full((NPAD,), G, jnp.int32).at[:N].set(batch.astype(jnp.int32))

    h = jnp.zeros((NPAD, D), x.dtype).at[:N].set(x)
    readouts = []
    for p in gin_params:
        agg = _make_sc_agg()(h, edges)
        h, ro = _tc_layer(h, agg, batch,
                          p["W1"], p["b1"].reshape(1, D),
                          p["W2"], p["b2"].reshape(1, D))
        readouts.append(ro)
    m1p = jnp.concatenate(readouts, axis=1)
    return (m1p, m1p)


# speedup vs baseline: 3.2680x; 1.0037x over previous
import functools

import jax
import jax.numpy as jnp
from jax import lax
from jax.experimental import pallas as pl
from jax.experimental.pallas import tpu as pltpu
from jax.experimental.pallas import tpu_sc as plsc

N = 10000
E = 320000
D = 128
G = 64
NC = 2
NS = 16
CHUNK = 128
EPW = 10240
NPADE = EPW - E // (NC * NS)
NHALF = 2
NCH = EPW // CHUNK // NHALF
NPAIR = NCH // 2
NPAD = 10240
RPW = NPAD // NS
ZROWS = 128


@functools.cache
def _make_sc_agg():
    mesh = plsc.VectorSubcoreMesh(core_axis_name="c", subcore_axis_name="s",
                                  num_cores=NC, num_subcores=NS)

    @functools.partial(
        pl.kernel,
        out_type=jax.ShapeDtypeStruct((NC, NPAD, D), jnp.float32),
        mesh=mesh,
        scratch_types=[
            pltpu.VMEM_SHARED((NPAD, D), jnp.float32),
            pltpu.VMEM((NCH, CHUNK), jnp.int32),
            pltpu.VMEM((NCH, CHUNK), jnp.int32),
            pltpu.VMEM((CHUNK, D), jnp.float32),
            pltpu.VMEM((CHUNK, D), jnp.float32),
            pltpu.SemaphoreType.DMA,
            pltpu.SemaphoreType.DMA,
        ],
    )
    def sc_agg(h_hbm, edges_hbm, out_hbm,
               acc_sh, src_v, dst_v, rows0_v, rows1_v, sem0, sem1):
        c = lax.axis_index("c")
        s = lax.axis_index("s")

        def _zrow(i, _):
            for k in range(D // 16):
                rows0_v[i, pl.ds(k * 16, 16)] = jnp.zeros((16,), jnp.float32)
            return 0
        lax.fori_loop(0, ZROWS, _zrow, 0)
        for t in range(RPW // ZROWS):
            pltpu.async_copy(rows0_v.at[pl.ds(0, ZROWS)],
                             acc_sh.at[pl.ds(s * RPW + t * ZROWS, ZROWS)],
                             sem1)
        pltpu.sync_copy(edges_hbm.at[0, c, s, 0], src_v)
        pltpu.sync_copy(edges_hbm.at[1, c, s, 0], dst_v)
        for t in range(RPW // ZROWS):
            pltpu.make_async_copy(
                rows0_v.at[pl.ds(0, ZROWS)],
                acc_sh.at[pl.ds(s * RPW + t * ZROWS, ZROWS)], sem1).wait()
        plsc.subcore_barrier()

        for half in range(NHALF):
            if half > 0:
                pltpu.sync_copy(edges_hbm.at[0, c, s, half], src_v)
                pltpu.sync_copy(edges_hbm.at[1, c, s, half], dst_v)
            pltpu.async_copy(h_hbm.at[src_v.at[0]], rows0_v, sem0)
            pltpu.async_copy(h_hbm.at[src_v.at[1]], rows1_v, sem1)

            def _pair(jj, _):
                j0 = 2 * jj
                pltpu.make_async_copy(h_hbm.at[src_v.at[j0]], rows0_v,
                                      sem0).wait()
                pltpu.sync_copy(rows0_v, acc_sh.at[dst_v.at[j0]], add=True)
                pltpu.async_copy(h_hbm.at[src_v.at[j0 + 2]], rows0_v, sem0)
                pltpu.make_async_copy(h_hbm.at[src_v.at[j0 + 1]], rows1_v,
                                      sem1).wait()
                pltpu.sync_copy(rows1_v, acc_sh.at[dst_v.at[j0 + 1]],
                                add=True)
                pltpu.async_copy(h_hbm.at[src_v.at[j0 + 3]], rows1_v, sem1)
                return 0
            lax.fori_loop(0, NPAIR - 1, _pair, 0)
            pltpu.make_async_copy(h_hbm.at[src_v.at[NCH - 2]], rows0_v,
                                  sem0).wait()
            pltpu.sync_copy(rows0_v, acc_sh.at[dst_v.at[NCH - 2]], add=True)
            pltpu.make_async_copy(h_hbm.at[src_v.at[NCH - 1]], rows1_v,
                                  sem1).wait()
            pltpu.sync_copy(rows1_v, acc_sh.at[dst_v.at[NCH - 1]], add=True)
        plsc.subcore_barrier()

        pltpu.sync_copy(acc_sh.at[pl.ds(s * RPW, RPW)],
                        out_hbm.at[c, pl.ds(s * RPW, RPW)])

    return sc_agg


ROWS_BLK = 2048


def _tc_layer_body(h_ref, a_ref, b_ref, w1_ref, b1_ref,
                   w2_ref, b2_ref, o_ref, ro_ref):
    hv = h_ref[...] + a_ref[0] + a_ref[1]
    t = jnp.maximum(
        jnp.dot(hv, w1_ref[...], preferred_element_type=jnp.float32)
        + b1_ref[...], 0.0)
    o = jnp.dot(t, w2_ref[...], preferred_element_type=jnp.float32) + b2_ref[...]
    hn = jnp.maximum(o, 0.0)
    rid = (lax.broadcasted_iota(jnp.int32, (ROWS_BLK, 1), 0)
           + pl.program_id(0) * ROWS_BLK)
    hn = jnp.where(rid < N, hn, 0.0)
    o_ref[...] = hn
    bb = b_ref[pl.ds(pl.program_id(0) * ROWS_BLK, ROWS_BLK)]
    sel = (lax.broadcasted_iota(jnp.int32, (G, ROWS_BLK), 0)
           == bb[None, :]).astype(jnp.float32)
    contrib = jnp.dot(sel, hn, preferred_element_type=jnp.float32)

    @pl.when(pl.program_id(0) == 0)
    def _():
        ro_ref[...] = contrib

    @pl.when(pl.program_id(0) != 0)
    def _():
        ro_ref[...] += contrib


def _tc_layer(h, agg, batch, w1, b1, w2, b2):
    nblk = NPAD // ROWS_BLK
    return pl.pallas_call(
        _tc_layer_body,
        grid=(nblk,),
        in_specs=[
            pl.BlockSpec((ROWS_BLK, D), lambda i: (i, 0)),
            pl.BlockSpec((NC, ROWS_BLK, D), lambda i: (0, i, 0)),
            pl.BlockSpec((NPAD,), lambda i: (0,)),
            pl.BlockSpec((D, D), lambda i: (0, 0)),
            pl.BlockSpec((1, D), lambda i: (0, 0)),
            pl.BlockSpec((D, D), lambda i: (0, 0)),
            pl.BlockSpec((1, D), lambda i: (0, 0)),
        ],
        out_specs=[
            pl.BlockSpec((ROWS_BLK, D), lambda i: (i, 0)),
            pl.BlockSpec((G, D), lambda i: (0, 0)),
        ],
        out_shape=[
            jax.ShapeDtypeStruct((NPAD, D), jnp.float32),
            jax.ShapeDtypeStruct((G, D), jnp.float32),
        ],
    )(h, agg, batch, w1, b1, w2, b2)


def kernel(x, edge_index, batch, device, gin_params, mlp_params, pool_params):
    del device, mlp_params, pool_params
    nw = NC * NS
    e3 = edge_index.astype(jnp.int32).reshape(2, nw, E // nw)
    wids = jnp.arange(nw, dtype=jnp.int32)[:, None]
    ii = jnp.arange(NPADE, dtype=jnp.int32)[None, :]
    pad_src = jnp.broadcast_to(N + (ii % (NPAD - N)), (nw, NPADE))
    pad_dst = jnp.broadcast_to(wids + nw * ii, (nw, NPADE))
    pads = jnp.stack([pad_src, pad_dst])
    edges = jnp.concatenate([e3, pads], axis=2).reshape(
        2, NC, NS, NHALF, NCH, CHUNK)
    batch = jnp.full((NPAD,), G, jnp.int32).at[:N].set(batch.astype(jnp.int32))

    h = jnp.zeros((NPAD, D), x.dtype).at[:N].set(x)
    readouts = []
    for p in gin_params:
        agg = _make_sc_agg()(h, edges)
        h, ro = _tc_layer(h, agg, batch,
                          p["W1"], p["b1"].reshape(1, D),
                          p["W2"], p["b2"].reshape(1, D))
        readouts.append(ro)
    m1p = jnp.concatenate(readouts, axis=1)
    return (m1p, m1p)
